# baseline (speedup 1.0000x reference)
import jax
import jax.numpy as jnp
from jax import lax
from jax.experimental import pallas as pl
from jax.experimental.pallas import tpu as pltpu
from jax.experimental.pallas import tpu_sc as plsc

D = 64
NC = 2
NS = 16
NW = NC * NS
LANES = 16
SCALE = 8.0
CHUNK = 512


def _body(idx_hbm, table_hbm, out_hbm, idx_v, rows_v, gsem):
    wid = lax.axis_index("s") * NC + lax.axis_index("c")
    n_total = idx_hbm.shape[0]
    per_w = n_total // NW
    iters = per_w // CHUNK
    base_w = wid * per_w

    @pl.loop(0, iters)
    def _chunk(g):
        base = base_w + g * CHUNK
        pltpu.sync_copy(idx_hbm.at[pl.ds(base, CHUNK)], idx_v)
        pltpu.async_copy(table_hbm.at[idx_v], rows_v, gsem).wait()

        @pl.loop(0, CHUNK)
        def _scale(r):
            for c in range(D // LANES):
                sl = (r, pl.ds(c * LANES, LANES))
                rows_v[sl] = rows_v[sl] * SCALE

        pltpu.sync_copy(rows_v, out_hbm.at[pl.ds(base, CHUNK)])


def kernel(x, W):
    B, H = x.shape
    n = B * H
    xf = x.reshape(n).astype(jnp.int32)
    mesh = plsc.VectorSubcoreMesh(core_axis_name="c", subcore_axis_name="s")
    out = pl.kernel(
        _body,
        out_type=jax.ShapeDtypeStruct((n, D), jnp.float32),
        mesh=mesh,
        scratch_types=[
            pltpu.VMEM((CHUNK,), jnp.int32),
            pltpu.VMEM((CHUNK, D), jnp.float32),
            pltpu.SemaphoreType.DMA,
        ],
        compiler_params=pltpu.CompilerParams(use_tc_tiling_on_sc=False),
    )(xf, W)
    return out.reshape(B, H, D)


# speedup vs baseline: 1.1386x; 1.1386x over previous
import jax
import jax.numpy as jnp
from jax import lax
from jax.experimental import pallas as pl
from jax.experimental.pallas import tpu as pltpu
from jax.experimental.pallas import tpu_sc as plsc

D = 64
NC = 2
NS = 16
NW = NC * NS
LANES = 16
SCALE = 8.0
CHUNK = 512


def _body(idx_hbm, table_hbm, out_hbm, idx_v, rows_v, gsem, ssem):
    wid = lax.axis_index("s") * NC + lax.axis_index("c")
    n_total = idx_hbm.shape[0]
    per_w = n_total // NW
    iters = per_w // CHUNK
    base_w = wid * per_w

    def load_idx(it, slot):
        pltpu.sync_copy(idx_hbm.at[pl.ds(base_w + it * CHUNK, CHUNK)],
                        idx_v.at[slot])

    def start_gather(slot):
        pltpu.async_copy(table_hbm.at[idx_v.at[slot]], rows_v.at[slot],
                         gsem.at[slot])

    def wait_gather(slot):
        pltpu.make_async_copy(table_hbm.at[idx_v.at[slot]], rows_v.at[slot],
                              gsem.at[slot]).wait()

    def start_store(it, slot):
        pltpu.async_copy(rows_v.at[slot],
                         out_hbm.at[pl.ds(base_w + it * CHUNK, CHUNK)],
                         ssem.at[slot])

    def wait_store(slot):
        pltpu.make_async_copy(rows_v.at[slot],
                              out_hbm.at[pl.ds(base_w, CHUNK)],
                              ssem.at[slot]).wait()

    load_idx(0, 0)
    start_gather(0)

    @pl.loop(0, iters, step=2)
    def _pair(g):
        for b in range(2):
            cur = g + b
            nxt = 1 - b

            @pl.when(cur + 1 < iters)
            def _prefetch():
                load_idx(cur + 1, nxt)

                @pl.when(cur >= 1)
                def _drain():
                    wait_store(nxt)

                start_gather(nxt)

            wait_gather(b)

            @pl.loop(0, CHUNK, unroll=4)
            def _scale(r):
                for c in range(D // LANES):
                    sl = (b, r, pl.ds(c * LANES, LANES))
                    rows_v[sl] = rows_v[sl] * SCALE

            start_store(cur, b)

    wait_store(0)
    wait_store(1)


def kernel(x, W):
    B, H = x.shape
    n = B * H
    xf = x.reshape(n).astype(jnp.int32)
    mesh = plsc.VectorSubcoreMesh(core_axis_name="c", subcore_axis_name="s")
    out = pl.kernel(
        _body,
        out_type=jax.ShapeDtypeStruct((n, D), jnp.float32),
        mesh=mesh,
        scratch_types=[
            pltpu.VMEM((2, CHUNK), jnp.int32),
            pltpu.VMEM((2, CHUNK, D), jnp.float32),
            pltpu.SemaphoreType.DMA((2,)),
            pltpu.SemaphoreType.DMA((2,)),
        ],
        compiler_params=pltpu.CompilerParams(use_tc_tiling_on_sc=False),
    )(xf, W)
    return out.reshape(B, H, D)


# speedup vs baseline: 1.3870x; 1.2182x over previous
import jax
import jax.numpy as jnp
from jax import lax
from jax.experimental import pallas as pl
from jax.experimental.pallas import tpu as pltpu
from jax.experimental.pallas import tpu_sc as plsc

D = 64
DP = 128
NC = 2
NS = 16
NW = NC * NS
LANES = 16
SCALE = 8.0
CHUNK = 256


def _body(idx_hbm, table_hbm, out_hbm,
          idx0, idx1, rows0, rows1, gsem, ssem):
    wid = lax.axis_index("s") * NC + lax.axis_index("c")
    n_total = idx_hbm.shape[0]
    per_w = n_total // NW
    iters = per_w // CHUNK
    base_w = wid * per_w

    idx_v = (idx0, idx1)
    rows_v = (rows0, rows1)

    def load_idx(it, slot):
        pltpu.sync_copy(idx_hbm.at[pl.ds(base_w + it * CHUNK, CHUNK)],
                        idx_v[slot])

    def start_gather(slot):
        pltpu.async_copy(table_hbm.at[idx_v[slot]], rows_v[slot],
                         gsem.at[slot])

    def wait_gather(slot):
        pltpu.make_async_copy(table_hbm.at[idx_v[slot]], rows_v[slot],
                              gsem.at[slot]).wait()

    def start_store(it, slot):
        pltpu.async_copy(rows_v[slot],
                         out_hbm.at[pl.ds(base_w + it * CHUNK, CHUNK)],
                         ssem.at[slot])

    def wait_store(slot):
        pltpu.make_async_copy(rows_v[slot],
                              out_hbm.at[pl.ds(base_w, CHUNK)],
                              ssem.at[slot]).wait()

    load_idx(0, 0)
    start_gather(0)

    @pl.loop(0, iters, step=2)
    def _pair(g):
        for b in range(2):
            cur = g + b
            nxt = 1 - b

            @pl.when(cur + 1 < iters)
            def _prefetch():
                load_idx(cur + 1, nxt)

                @pl.when(cur >= 1)
                def _drain():
                    wait_store(nxt)

                start_gather(nxt)

            wait_gather(b)

            @pl.loop(0, CHUNK, unroll=4)
            def _scale(r):
                for c in range(D // LANES):
                    sl = (r, pl.ds(c * LANES, LANES))
                    rows_v[b][sl] = rows_v[b][sl] * SCALE

            start_store(cur, b)

    wait_store(0)
    wait_store(1)


def kernel(x, W):
    B, H = x.shape
    n = B * H
    xf = x.reshape(n).astype(jnp.int32)
    Wp = jnp.pad(W, ((0, 0), (0, DP - D)))
    mesh = plsc.VectorSubcoreMesh(core_axis_name="c", subcore_axis_name="s")
    out = pl.kernel(
        _body,
        out_type=jax.ShapeDtypeStruct((n, DP), jnp.float32),
        mesh=mesh,
        scratch_types=[
            pltpu.VMEM((CHUNK,), jnp.int32),
            pltpu.VMEM((CHUNK,), jnp.int32),
            pltpu.VMEM((CHUNK, DP), jnp.float32),
            pltpu.VMEM((CHUNK, DP), jnp.float32),
            pltpu.SemaphoreType.DMA((2,)),
            pltpu.SemaphoreType.DMA((2,)),
        ],
        compiler_params=pltpu.CompilerParams(use_tc_tiling_on_sc=True),
    )(xf, Wp)
    return out[:, :D].reshape(B, H, D)


# speedup vs baseline: 1.3907x; 1.0027x over previous
import jax
import jax.numpy as jnp
from jax import lax
from jax.experimental import pallas as pl
from jax.experimental.pallas import tpu as pltpu
from jax.experimental.pallas import tpu_sc as plsc

D = 64
DP = 128
NC = 2
NS = 16
NW = NC * NS
LANES = 16
SCALE = 8.0
CHUNK = 320


def _body(idx_hbm, table_hbm, out_hbm,
          idx0, idx1, rows0, rows1, gsem, ssem):
    wid = lax.axis_index("s") * NC + lax.axis_index("c")
    n_total = idx_hbm.shape[0]
    per_w = n_total // NW
    iters = per_w // CHUNK
    base_w = wid * per_w

    idx_v = (idx0, idx1)
    rows_v = (rows0, rows1)

    def load_idx(it, slot):
        pltpu.sync_copy(idx_hbm.at[pl.ds(base_w + it * CHUNK, CHUNK)],
                        idx_v[slot])

    def start_gather(slot):
        pltpu.async_copy(table_hbm.at[idx_v[slot]], rows_v[slot],
                         gsem.at[slot])

    def wait_gather(slot):
        pltpu.make_async_copy(table_hbm.at[idx_v[slot]], rows_v[slot],
                              gsem.at[slot]).wait()

    def start_store(it, slot):
        pltpu.async_copy(rows_v[slot],
                         out_hbm.at[pl.ds(base_w + it * CHUNK, CHUNK)],
                         ssem.at[slot])

    def wait_store(slot):
        pltpu.make_async_copy(rows_v[slot],
                              out_hbm.at[pl.ds(base_w, CHUNK)],
                              ssem.at[slot]).wait()

    load_idx(0, 0)
    start_gather(0)

    @pl.loop(0, iters, step=2)
    def _pair(g):
        for b in range(2):
            cur = g + b
            nxt = 1 - b

            @pl.when(cur + 1 < iters)
            def _prefetch():
                load_idx(cur + 1, nxt)

                @pl.when(cur >= 1)
                def _drain():
                    wait_store(nxt)

                start_gather(nxt)

            wait_gather(b)

            @pl.loop(0, CHUNK, unroll=4)
            def _scale(r):
                for c in range(D // LANES):
                    sl = (r, pl.ds(c * LANES, LANES))
                    rows_v[b][sl] = rows_v[b][sl] * SCALE

            start_store(cur, b)

    wait_store(0)
    wait_store(1)


def kernel(x, W):
    B, H = x.shape
    n = B * H
    xf = x.reshape(n).astype(jnp.int32)
    Wp = jnp.pad(W, ((0, 0), (0, DP - D)))
    mesh = plsc.VectorSubcoreMesh(core_axis_name="c", subcore_axis_name="s")
    out = pl.kernel(
        _body,
        out_type=jax.ShapeDtypeStruct((n, DP), jnp.float32),
        mesh=mesh,
        scratch_types=[
            pltpu.VMEM((CHUNK,), jnp.int32),
            pltpu.VMEM((CHUNK,), jnp.int32),
            pltpu.VMEM((CHUNK, DP), jnp.float32),
            pltpu.VMEM((CHUNK, DP), jnp.float32),
            pltpu.SemaphoreType.DMA((2,)),
            pltpu.SemaphoreType.DMA((2,)),
        ],
        compiler_params=pltpu.CompilerParams(use_tc_tiling_on_sc=True),
    )(xf, Wp)
    return out[:, :D].reshape(B, H, D)
